# Initial kernel scaffold; baseline (speedup 1.0000x reference)
import jax
import jax.numpy as jnp
from jax.experimental import pallas as pl


def kernel(x, user_embedding, SW1, Sb1, SW2, Sb2, EW1, Eb1, EW2, Eb2, UW1, Ub1, UW2, Ub2):
    raise NotImplementedError("write your pallas kernel here")


# speedup vs baseline: 8.4310x; 8.4310x over previous
import functools

import jax
import jax.numpy as jnp
from jax import lax
from jax.experimental import pallas as pl

B, D, E = 4096, 64, 8
S1, S2 = 32, 8
BLK = 512


def _moe_kernel(x_ref, ue_ref, sw1t_ref, sb1_ref, sw2t_ref, sb2_ref,
                w1cat_ref, b1cat_ref, w2stack_ref, eb2_ref,
                uw1t_ref, ub1_ref, uw2t_ref, ub2_ref, out_ref):
    xb = x_ref[...]
    ue = ue_ref[...]

    h = jnp.maximum(
        jnp.dot(ue, sw1t_ref[...], preferred_element_type=jnp.float32)
        + sb1_ref[...], 0.0)
    logits = (jnp.dot(h, sw2t_ref[...], preferred_element_type=jnp.float32)
              + sb2_ref[...])
    routes = jnp.argmax(logits, axis=-1).reshape(BLK, 1)

    eidx = lax.broadcasted_iota(jnp.int32, (BLK, E), 1)
    onehot = (eidx == routes).astype(jnp.float32)
    colidx = lax.broadcasted_iota(jnp.int32, (BLK, E * D), 1) // D
    maskfull = (colidx == routes).astype(jnp.float32)

    h1 = jnp.maximum(
        jnp.dot(xb, w1cat_ref[...], preferred_element_type=jnp.float32)
        + b1cat_ref[...], 0.0)
    h1m = h1 * maskfull
    out = (jnp.dot(h1m, w2stack_ref[...], preferred_element_type=jnp.float32)
           + jnp.dot(onehot, eb2_ref[...], preferred_element_type=jnp.float32))

    uh = jnp.maximum(
        jnp.dot(xb, uw1t_ref[...], preferred_element_type=jnp.float32)
        + ub1_ref[...], 0.0)
    out = out + jnp.dot(uh, uw2t_ref[...],
                        preferred_element_type=jnp.float32) + ub2_ref[...]

    out_ref[...] = out


@jax.jit
def kernel(x, user_embedding, SW1, Sb1, SW2, Sb2, EW1, Eb1, EW2, Eb2,
           UW1, Ub1, UW2, Ub2):
    w1cat = jnp.transpose(EW1, (2, 0, 1)).reshape(D, E * D)
    b1cat = Eb1.reshape(1, E * D)
    w2stack = jnp.transpose(EW2, (0, 2, 1)).reshape(E * D, D)

    tok = lambda i: (i, 0)
    const = lambda i: (0, 0)
    out = pl.pallas_call(
        _moe_kernel,
        grid=(B // BLK,),
        in_specs=[
            pl.BlockSpec((BLK, D), tok),
            pl.BlockSpec((BLK, D), tok),
            pl.BlockSpec((D, S1), const),
            pl.BlockSpec((1, S1), const),
            pl.BlockSpec((S1, S2), const),
            pl.BlockSpec((1, S2), const),
            pl.BlockSpec((D, E * D), const),
            pl.BlockSpec((1, E * D), const),
            pl.BlockSpec((E * D, D), const),
            pl.BlockSpec((E, D), const),
            pl.BlockSpec((D, D), const),
            pl.BlockSpec((1, D), const),
            pl.BlockSpec((D, D), const),
            pl.BlockSpec((1, D), const),
        ],
        out_specs=pl.BlockSpec((BLK, D), tok),
        out_shape=jax.ShapeDtypeStruct((B, D), jnp.float32),
    )(x, user_embedding, SW1.T, Sb1.reshape(1, S1), SW2.T,
      Sb2.reshape(1, S2), w1cat, b1cat, w2stack, Eb2,
      UW1.T, Ub1.reshape(1, D), UW2.T, Ub2.reshape(1, D))
    return out


# speedup vs baseline: 9.1333x; 1.0833x over previous
import functools

import jax
import jax.numpy as jnp
from jax import lax
from jax.experimental import pallas as pl

B, D, E = 4096, 64, 8
S1, S2 = 32, 8
BLK = 1024


def _moe_kernel(x_ref, ue_ref, sw1t_ref, sb1_ref, sw2t_ref, sb2_ref,
                w1cat_ref, b1cat_ref, w2stack_ref, eb2_ref,
                uw1t_ref, ub1_ref, uw2t_ref, ub2_ref, out_ref):
    xb = x_ref[...]
    ue = ue_ref[...]

    h = jnp.maximum(
        jnp.dot(ue, sw1t_ref[...], preferred_element_type=jnp.float32)
        + sb1_ref[...], 0.0)
    logits = (jnp.dot(h, sw2t_ref[...], preferred_element_type=jnp.float32)
              + sb2_ref[...])
    routes = jnp.argmax(logits, axis=-1).reshape(BLK, 1)

    eidx = lax.broadcasted_iota(jnp.int32, (BLK, E), 1)
    onehot = (eidx == routes).astype(jnp.float32)
    colidx = lax.broadcasted_iota(jnp.int32, (BLK, E * D), 1) // D
    maskfull = (colidx == routes).astype(jnp.float32)

    h1 = jnp.maximum(
        jnp.dot(xb, w1cat_ref[...], preferred_element_type=jnp.float32)
        + b1cat_ref[...], 0.0)
    h1m = h1 * maskfull
    out = (jnp.dot(h1m, w2stack_ref[...], preferred_element_type=jnp.float32)
           + jnp.dot(onehot, eb2_ref[...], preferred_element_type=jnp.float32))

    uh = jnp.maximum(
        jnp.dot(xb, uw1t_ref[...], preferred_element_type=jnp.float32)
        + ub1_ref[...], 0.0)
    out = out + jnp.dot(uh, uw2t_ref[...],
                        preferred_element_type=jnp.float32) + ub2_ref[...]

    out_ref[...] = out


@jax.jit
def kernel(x, user_embedding, SW1, Sb1, SW2, Sb2, EW1, Eb1, EW2, Eb2,
           UW1, Ub1, UW2, Ub2):
    w1cat = jnp.transpose(EW1, (2, 0, 1)).reshape(D, E * D)
    b1cat = Eb1.reshape(1, E * D)
    w2stack = jnp.transpose(EW2, (0, 2, 1)).reshape(E * D, D)

    tok = lambda i: (i, 0)
    const = lambda i: (0, 0)
    out = pl.pallas_call(
        _moe_kernel,
        grid=(B // BLK,),
        in_specs=[
            pl.BlockSpec((BLK, D), tok),
            pl.BlockSpec((BLK, D), tok),
            pl.BlockSpec((D, S1), const),
            pl.BlockSpec((1, S1), const),
            pl.BlockSpec((S1, S2), const),
            pl.BlockSpec((1, S2), const),
            pl.BlockSpec((D, E * D), const),
            pl.BlockSpec((1, E * D), const),
            pl.BlockSpec((E * D, D), const),
            pl.BlockSpec((E, D), const),
            pl.BlockSpec((D, D), const),
            pl.BlockSpec((1, D), const),
            pl.BlockSpec((D, D), const),
            pl.BlockSpec((1, D), const),
        ],
        out_specs=pl.BlockSpec((BLK, D), tok),
        out_shape=jax.ShapeDtypeStruct((B, D), jnp.float32),
    )(x, user_embedding, SW1.T, Sb1.reshape(1, S1), SW2.T,
      Sb2.reshape(1, S2), w1cat, b1cat, w2stack, Eb2,
      UW1.T, Ub1.reshape(1, D), UW2.T, Ub2.reshape(1, D))
    return out


# speedup vs baseline: 9.1854x; 1.0057x over previous
import functools

import jax
import jax.numpy as jnp
from jax import lax
from jax.experimental import pallas as pl

B, D, E = 4096, 64, 8
S1, S2 = 32, 8
BLK = 2048


def _moe_kernel(x_ref, ue_ref, sw1t_ref, sb1_ref, sw2t_ref, sb2_ref,
                w1cat_ref, b1cat_ref, w2stack_ref, eb2_ref,
                uw1t_ref, ub1_ref, uw2t_ref, ub2_ref, out_ref):
    xb = x_ref[...]
    ue = ue_ref[...]

    h = jnp.maximum(
        jnp.dot(ue, sw1t_ref[...], preferred_element_type=jnp.float32)
        + sb1_ref[...], 0.0)
    logits = (jnp.dot(h, sw2t_ref[...], preferred_element_type=jnp.float32)
              + sb2_ref[...])
    routes = jnp.argmax(logits, axis=-1).reshape(BLK, 1)

    eidx = lax.broadcasted_iota(jnp.int32, (BLK, E), 1)
    onehot = (eidx == routes).astype(jnp.float32)
    colidx = lax.broadcasted_iota(jnp.int32, (BLK, E * D), 1) // D
    maskfull = (colidx == routes).astype(jnp.float32)

    h1 = jnp.maximum(
        jnp.dot(xb, w1cat_ref[...], preferred_element_type=jnp.float32)
        + b1cat_ref[...], 0.0)
    h1m = h1 * maskfull
    out = (jnp.dot(h1m, w2stack_ref[...], preferred_element_type=jnp.float32)
           + jnp.dot(onehot, eb2_ref[...], preferred_element_type=jnp.float32))

    uh = jnp.maximum(
        jnp.dot(xb, uw1t_ref[...], preferred_element_type=jnp.float32)
        + ub1_ref[...], 0.0)
    out = out + jnp.dot(uh, uw2t_ref[...],
                        preferred_element_type=jnp.float32) + ub2_ref[...]

    out_ref[...] = out


@jax.jit
def kernel(x, user_embedding, SW1, Sb1, SW2, Sb2, EW1, Eb1, EW2, Eb2,
           UW1, Ub1, UW2, Ub2):
    w1cat = jnp.transpose(EW1, (2, 0, 1)).reshape(D, E * D)
    b1cat = Eb1.reshape(1, E * D)
    w2stack = jnp.transpose(EW2, (0, 2, 1)).reshape(E * D, D)

    tok = lambda i: (i, 0)
    const = lambda i: (0, 0)
    out = pl.pallas_call(
        _moe_kernel,
        grid=(B // BLK,),
        in_specs=[
            pl.BlockSpec((BLK, D), tok),
            pl.BlockSpec((BLK, D), tok),
            pl.BlockSpec((D, S1), const),
            pl.BlockSpec((1, S1), const),
            pl.BlockSpec((S1, S2), const),
            pl.BlockSpec((1, S2), const),
            pl.BlockSpec((D, E * D), const),
            pl.BlockSpec((1, E * D), const),
            pl.BlockSpec((E * D, D), const),
            pl.BlockSpec((E, D), const),
            pl.BlockSpec((D, D), const),
            pl.BlockSpec((1, D), const),
            pl.BlockSpec((D, D), const),
            pl.BlockSpec((1, D), const),
        ],
        out_specs=pl.BlockSpec((BLK, D), tok),
        out_shape=jax.ShapeDtypeStruct((B, D), jnp.float32),
    )(x, user_embedding, SW1.T, Sb1.reshape(1, S1), SW2.T,
      Sb2.reshape(1, S2), w1cat, b1cat, w2stack, Eb2,
      UW1.T, Ub1.reshape(1, D), UW2.T, Ub2.reshape(1, D))
    return out


# speedup vs baseline: 12.8820x; 1.4024x over previous
import jax
import jax.numpy as jnp
from jax import lax
from jax.experimental import pallas as pl

B, D, E = 4096, 64, 8
S1, S2 = 32, 8
BLK = 2048

_DN_T = (((1,), (1,)), ((), ()))


def _dot_t(a, w):
    return lax.dot_general(a, w, _DN_T, preferred_element_type=jnp.float32)


def _moe_kernel(x_ref, ue_ref, sw1_ref, sb1_ref, sw2_ref, sb2_ref,
                w1r_ref, b1cat_ref, w2stack_ref, eb2_ref,
                uw1_ref, ub1_ref, uw2_ref, ub2_ref, out_ref):
    xb = x_ref[...]
    ue = ue_ref[...]

    h = jnp.maximum(_dot_t(ue, sw1_ref[...]) + sb1_ref[...], 0.0)
    logits = _dot_t(h, sw2_ref[...]) + sb2_ref[...]
    routes = jnp.argmax(logits, axis=-1).reshape(BLK, 1)

    eidx = lax.broadcasted_iota(jnp.int32, (BLK, E), 1)
    onehot = (eidx == routes).astype(jnp.float32)
    colidx = lax.broadcasted_iota(jnp.int32, (BLK, E * D), 1) // D
    maskfull = (colidx == routes).astype(jnp.float32)

    h1 = jnp.maximum(_dot_t(xb, w1r_ref[...]) + b1cat_ref[...], 0.0)
    h1m = h1 * maskfull
    out = (jnp.dot(h1m, w2stack_ref[...], preferred_element_type=jnp.float32)
           + jnp.dot(onehot, eb2_ref[...], preferred_element_type=jnp.float32))

    uh = jnp.maximum(_dot_t(xb, uw1_ref[...]) + ub1_ref[...], 0.0)
    out = out + _dot_t(uh, uw2_ref[...]) + ub2_ref[...]

    out_ref[...] = out


@jax.jit
def kernel(x, user_embedding, SW1, Sb1, SW2, Sb2, EW1, Eb1, EW2, Eb2,
           UW1, Ub1, UW2, Ub2):
    w1r = EW1.reshape(E * D, D)
    b1cat = Eb1.reshape(1, E * D)
    w2stack = jnp.transpose(EW2, (0, 2, 1)).reshape(E * D, D)

    tok = lambda i: (i, 0)
    const = lambda i: (0, 0)
    out = pl.pallas_call(
        _moe_kernel,
        grid=(B // BLK,),
        in_specs=[
            pl.BlockSpec((BLK, D), tok),
            pl.BlockSpec((BLK, D), tok),
            pl.BlockSpec((S1, D), const),
            pl.BlockSpec((1, S1), const),
            pl.BlockSpec((S2, S1), const),
            pl.BlockSpec((1, S2), const),
            pl.BlockSpec((E * D, D), const),
            pl.BlockSpec((1, E * D), const),
            pl.BlockSpec((E * D, D), const),
            pl.BlockSpec((E, D), const),
            pl.BlockSpec((D, D), const),
            pl.BlockSpec((1, D), const),
            pl.BlockSpec((D, D), const),
            pl.BlockSpec((1, D), const),
        ],
        out_specs=pl.BlockSpec((BLK, D), tok),
        out_shape=jax.ShapeDtypeStruct((B, D), jnp.float32),
    )(x, user_embedding, SW1, Sb1.reshape(1, S1), SW2,
      Sb2.reshape(1, S2), w1r, b1cat, w2stack, Eb2,
      UW1, Ub1.reshape(1, D), UW2, Ub2.reshape(1, D))
    return out


# speedup vs baseline: 14.3286x; 1.1123x over previous
import jax
import jax.numpy as jnp
from jax import lax
from jax.experimental import pallas as pl

B, D, E = 4096, 64, 8
S1, S2 = 32, 8
BLK = 2048

_DN_T = (((1,), (1,)), ((), ()))


def _dot_t(a, w):
    return lax.dot_general(a, w, _DN_T, preferred_element_type=jnp.float32)


def _moe_kernel(x_ref, ue_ref, sw1_ref, sb1_ref, sw2_ref, sb2_ref,
                w1r_ref, b1cat_ref, w2r_ref, eb2_ref,
                uw1_ref, ub1_ref, uw2_ref, ub2_ref, out_ref):
    xb = x_ref[...]
    ue = ue_ref[...]

    h = jnp.maximum(_dot_t(ue, sw1_ref[...]) + sb1_ref[...], 0.0)
    logits = _dot_t(h, sw2_ref[...]) + sb2_ref[...]
    routes = jnp.argmax(logits, axis=-1).reshape(BLK, 1)

    eidx = lax.broadcasted_iota(jnp.int32, (BLK, E), 1)
    onehot = (eidx == routes).astype(jnp.float32)
    colidx = lax.broadcasted_iota(jnp.int32, (BLK, E * D), 1) // D
    maskfull = (colidx == routes).astype(jnp.float32)

    h1 = jnp.maximum(_dot_t(xb, w1r_ref[...]) + b1cat_ref[...], 0.0)
    h1m = h1 * maskfull

    w2stack = jnp.transpose(w2r_ref[...].reshape(E, D, D),
                            (0, 2, 1)).reshape(E * D, D)
    out = (jnp.dot(h1m, w2stack, preferred_element_type=jnp.float32)
           + jnp.dot(onehot, eb2_ref[...], preferred_element_type=jnp.float32))

    uh = jnp.maximum(_dot_t(xb, uw1_ref[...]) + ub1_ref[...], 0.0)
    out = out + _dot_t(uh, uw2_ref[...]) + ub2_ref[...]

    out_ref[...] = out


@jax.jit
def kernel(x, user_embedding, SW1, Sb1, SW2, Sb2, EW1, Eb1, EW2, Eb2,
           UW1, Ub1, UW2, Ub2):
    w1r = EW1.reshape(E * D, D)
    b1cat = Eb1.reshape(1, E * D)
    w2r = EW2.reshape(E * D, D)

    tok = lambda i: (i, 0)
    const = lambda i: (0, 0)
    out = pl.pallas_call(
        _moe_kernel,
        grid=(B // BLK,),
        in_specs=[
            pl.BlockSpec((BLK, D), tok),
            pl.BlockSpec((BLK, D), tok),
            pl.BlockSpec((S1, D), const),
            pl.BlockSpec((1, S1), const),
            pl.BlockSpec((S2, S1), const),
            pl.BlockSpec((1, S2), const),
            pl.BlockSpec((E * D, D), const),
            pl.BlockSpec((1, E * D), const),
            pl.BlockSpec((E * D, D), const),
            pl.BlockSpec((E, D), const),
            pl.BlockSpec((D, D), const),
            pl.BlockSpec((1, D), const),
            pl.BlockSpec((D, D), const),
            pl.BlockSpec((1, D), const),
        ],
        out_specs=pl.BlockSpec((BLK, D), tok),
        out_shape=jax.ShapeDtypeStruct((B, D), jnp.float32),
    )(x, user_embedding, SW1, Sb1.reshape(1, S1), SW2,
      Sb2.reshape(1, S2), w1r, b1cat, w2r, Eb2,
      UW1, Ub1.reshape(1, D), UW2, Ub2.reshape(1, D))
    return out
